# Initial kernel scaffold; baseline (speedup 1.0000x reference)
import jax
import jax.numpy as jnp
from jax.experimental import pallas as pl


def kernel(node_attr, edge_attr, edge_index, W, b):
    raise NotImplementedError("write your pallas kernel here")


# speedup vs baseline: 1.6718x; 1.6718x over previous
import functools

import jax
import jax.numpy as jnp
from jax import lax
from jax.experimental import pallas as pl
from jax.experimental.pallas import tpu as pltpu
from jax.experimental.pallas import tpu_sc as plsc

N_NODES = 10000
N_EDGES = 320000
D_FEAT = 128
D_EDGE = 16
D_HID = 128

_NC = 2
_NS = 16
_NW = _NC * _NS
_EPW = N_EDGES // _NW
_C = 80
_NCHUNK = _EPW // _C


def _proj_body(node_ref, ws_ref, wr_ref, ps_ref, pr_ref):
    x = node_ref[...]
    ps_ref[...] = jnp.dot(x, ws_ref[...], preferred_element_type=jnp.float32)
    pr_ref[...] = jnp.dot(x, wr_ref[...], preferred_element_type=jnp.float32)


def _edge_body(e_ref, we_ref, b_ref, t_ref):
    t_ref[...] = (
        jnp.dot(e_ref[...], we_ref[...], preferred_element_type=jnp.float32)
        + b_ref[...]
    )


def _sc_body(ps_hbm, pr_hbm, t_hbm, si_hbm, ri_hbm, out_hbm,
             si_v, ri_v, rs_v, rr_v, tt_v, sem_s, sem_r, sem_t):
    wid = lax.axis_index("s") * _NC + lax.axis_index("c")
    base = wid * _EPW

    def step(ci, carry):
        off = base + ci * _C
        pltpu.sync_copy(si_hbm.at[pl.ds(off, _C)], si_v)
        pltpu.sync_copy(ri_hbm.at[pl.ds(off, _C)], ri_v)
        g_s = pltpu.async_copy(ps_hbm.at[si_v], rs_v, sem_s)
        g_r = pltpu.async_copy(pr_hbm.at[ri_v], rr_v, sem_r)
        g_t = pltpu.async_copy(t_hbm.at[pl.ds(off, _C)], tt_v, sem_t)
        g_s.wait()
        g_r.wait()
        g_t.wait()

        def row(i, c):
            for k in range(D_HID // 16):
                sl = pl.ds(k * 16, 16)
                v = rs_v[i, sl] + rr_v[i, sl] + tt_v[i, sl]
                rs_v[i, sl] = jnp.maximum(v, 0.0)
            return c

        lax.fori_loop(0, _C, row, 0, unroll=2)
        pltpu.sync_copy(rs_v, out_hbm.at[pl.ds(off, _C)])
        return carry

    lax.fori_loop(0, _NCHUNK, step, 0)


def kernel(node_attr, edge_attr, edge_index, W, b):
    senders = edge_index[0].astype(jnp.int32)
    receivers = edge_index[1].astype(jnp.int32)
    w_s = W[:D_FEAT]
    w_r = W[D_FEAT:2 * D_FEAT]
    w_e = W[2 * D_FEAT:]
    b2 = b.reshape(1, D_HID)

    bm = 2000
    p_s, p_r = pl.pallas_call(
        _proj_body,
        grid=(N_NODES // bm,),
        in_specs=[
            pl.BlockSpec((bm, D_FEAT), lambda i: (i, 0)),
            pl.BlockSpec((D_FEAT, D_HID), lambda i: (0, 0)),
            pl.BlockSpec((D_FEAT, D_HID), lambda i: (0, 0)),
        ],
        out_specs=[
            pl.BlockSpec((bm, D_HID), lambda i: (i, 0)),
            pl.BlockSpec((bm, D_HID), lambda i: (i, 0)),
        ],
        out_shape=[
            jax.ShapeDtypeStruct((N_NODES, D_HID), jnp.float32),
            jax.ShapeDtypeStruct((N_NODES, D_HID), jnp.float32),
        ],
    )(node_attr, w_s, w_r)

    bme = 6400
    t_edge = pl.pallas_call(
        _edge_body,
        grid=(N_EDGES // bme,),
        in_specs=[
            pl.BlockSpec((bme, D_EDGE), lambda i: (i, 0)),
            pl.BlockSpec((D_EDGE, D_HID), lambda i: (0, 0)),
            pl.BlockSpec((1, D_HID), lambda i: (0, 0)),
        ],
        out_specs=pl.BlockSpec((bme, D_HID), lambda i: (i, 0)),
        out_shape=jax.ShapeDtypeStruct((N_EDGES, D_HID), jnp.float32),
    )(edge_attr, w_e, b2)

    sc_call = pl.kernel(
        _sc_body,
        out_type=jax.ShapeDtypeStruct((N_EDGES, D_HID), jnp.float32),
        mesh=plsc.VectorSubcoreMesh(core_axis_name="c", subcore_axis_name="s"),
        scratch_types=[
            pltpu.VMEM((_C,), jnp.int32),
            pltpu.VMEM((_C,), jnp.int32),
            pltpu.VMEM((_C, D_HID), jnp.float32),
            pltpu.VMEM((_C, D_HID), jnp.float32),
            pltpu.VMEM((_C, D_HID), jnp.float32),
            pltpu.SemaphoreType.DMA,
            pltpu.SemaphoreType.DMA,
            pltpu.SemaphoreType.DMA,
        ],
    )
    return sc_call(p_s, p_r, t_edge, senders, receivers)


# speedup vs baseline: 2.3955x; 1.4329x over previous
import jax
import jax.numpy as jnp
from jax import lax
from jax.experimental import pallas as pl
from jax.experimental.pallas import tpu as pltpu
from jax.experimental.pallas import tpu_sc as plsc

N_NODES = 10000
N_EDGES = 320000
D_FEAT = 128
D_EDGE = 16
D_HID = 128

_NC = 2
_NS = 16
_NW = _NC * _NS
_EPW = N_EDGES // _NW
_G = 80
_NCHUNK = _EPW // _G


def _proj_body(node_ref, ws_ref, wr_ref, ps_ref, pr_ref):
    x = node_ref[...]
    ps_ref[...] = jnp.dot(x, ws_ref[...], preferred_element_type=jnp.float32)
    pr_ref[...] = jnp.dot(x, wr_ref[...], preferred_element_type=jnp.float32)


def _edge_body(e_ref, we_ref, b_ref, t_ref):
    t_ref[...] = (
        jnp.dot(e_ref[...], we_ref[...], preferred_element_type=jnp.float32)
        + b_ref[...]
    )


def _sc_body(ps_hbm, pr_hbm, t_hbm, si_hbm, ri_hbm, out_hbm,
             si_all, ri_all, rs_v, rr_v, tt_v, ob_v,
             sem_s0, sem_s1, sem_r0, sem_r1, sem_t0, sem_t1,
             sem_o0, sem_o1, sem_i):
    gsem_s = (sem_s0, sem_s1)
    gsem_r = (sem_r0, sem_r1)
    gsem_t = (sem_t0, sem_t1)
    osem = (sem_o0, sem_o1)
    wid = lax.axis_index("s") * _NC + lax.axis_index("c")
    base = wid * _EPW

    ci1 = pltpu.async_copy(si_hbm.at[pl.ds(base, _EPW)], si_all, sem_i)
    ci2 = pltpu.async_copy(ri_hbm.at[pl.ds(base, _EPW)], ri_all, sem_i)
    ci1.wait()
    ci2.wait()

    def issue(c, b):
        loc = c * _G
        g_s = pltpu.async_copy(
            ps_hbm.at[si_all.at[pl.ds(loc, _G)]], rs_v.at[b], gsem_s[b])
        g_r = pltpu.async_copy(
            pr_hbm.at[ri_all.at[pl.ds(loc, _G)]], rr_v.at[b], gsem_r[b])
        g_t = pltpu.async_copy(
            t_hbm.at[pl.ds(base + loc, _G)], tt_v.at[b], gsem_t[b])
        del g_s, g_r, g_t

    def drain(b):
        pltpu.make_async_copy(
            t_hbm.at[pl.ds(0, _G)], rs_v.at[b], gsem_s[b]).wait()
        pltpu.make_async_copy(
            t_hbm.at[pl.ds(0, _G)], rr_v.at[b], gsem_r[b]).wait()
        pltpu.make_async_copy(
            t_hbm.at[pl.ds(0, _G)], tt_v.at[b], gsem_t[b]).wait()

    def wait_store(b):
        pltpu.make_async_copy(
            ob_v.at[b], out_hbm.at[pl.ds(0, _G)], osem[b]).wait()

    def compute(b):
        def row(i, carry):
            for k in range(D_HID // 16):
                sl = pl.ds(k * 16, 16)
                v = rs_v[b, i, sl] + rr_v[b, i, sl] + tt_v[b, i, sl]
                ob_v[b, i, sl] = jnp.maximum(v, 0.0)
            return carry
        lax.fori_loop(0, _G, row, 0, unroll=2)

    def process(c, b):
        drain(b)

        @pl.when(c >= 2)
        def _():
            wait_store(b)

        compute(b)
        st = pltpu.async_copy(
            ob_v.at[b], out_hbm.at[pl.ds(base + c * _G, _G)], osem[b])
        del st

        @pl.when(c + 2 < _NCHUNK)
        def _():
            issue(c + 2, b)

    issue(0, 0)
    issue(1, 1)

    def pair(g, carry):
        process(2 * g, 0)
        process(2 * g + 1, 1)
        return carry

    lax.fori_loop(0, _NCHUNK // 2, pair, 0)
    process(_NCHUNK - 1, 0)
    wait_store(0)
    wait_store(1)


def kernel(node_attr, edge_attr, edge_index, W, b):
    senders = edge_index[0].astype(jnp.int32)
    receivers = edge_index[1].astype(jnp.int32)
    w_s = W[:D_FEAT]
    w_r = W[D_FEAT:2 * D_FEAT]
    w_e = W[2 * D_FEAT:]
    b2 = b.reshape(1, D_HID)

    bm = 2000
    p_s, p_r = pl.pallas_call(
        _proj_body,
        grid=(N_NODES // bm,),
        in_specs=[
            pl.BlockSpec((bm, D_FEAT), lambda i: (i, 0)),
            pl.BlockSpec((D_FEAT, D_HID), lambda i: (0, 0)),
            pl.BlockSpec((D_FEAT, D_HID), lambda i: (0, 0)),
        ],
        out_specs=[
            pl.BlockSpec((bm, D_HID), lambda i: (i, 0)),
            pl.BlockSpec((bm, D_HID), lambda i: (i, 0)),
        ],
        out_shape=[
            jax.ShapeDtypeStruct((N_NODES, D_HID), jnp.float32),
            jax.ShapeDtypeStruct((N_NODES, D_HID), jnp.float32),
        ],
    )(node_attr, w_s, w_r)

    bme = 6400
    t_edge = pl.pallas_call(
        _edge_body,
        grid=(N_EDGES // bme,),
        in_specs=[
            pl.BlockSpec((bme, D_EDGE), lambda i: (i, 0)),
            pl.BlockSpec((D_EDGE, D_HID), lambda i: (0, 0)),
            pl.BlockSpec((1, D_HID), lambda i: (0, 0)),
        ],
        out_specs=pl.BlockSpec((bme, D_HID), lambda i: (i, 0)),
        out_shape=jax.ShapeDtypeStruct((N_EDGES, D_HID), jnp.float32),
    )(edge_attr, w_e, b2)

    sc_call = pl.kernel(
        _sc_body,
        out_type=jax.ShapeDtypeStruct((N_EDGES, D_HID), jnp.float32),
        mesh=plsc.VectorSubcoreMesh(core_axis_name="c", subcore_axis_name="s"),
        scratch_types=[
            pltpu.VMEM((_EPW,), jnp.int32),
            pltpu.VMEM((_EPW,), jnp.int32),
            pltpu.VMEM((2, _G, D_HID), jnp.float32),
            pltpu.VMEM((2, _G, D_HID), jnp.float32),
            pltpu.VMEM((2, _G, D_HID), jnp.float32),
            pltpu.VMEM((2, _G, D_HID), jnp.float32),
        ] + [pltpu.SemaphoreType.DMA] * 9,
    )
    return sc_call(p_s, p_r, t_edge, senders, receivers)
